# Initial kernel scaffold; baseline (speedup 1.0000x reference)
import jax
import jax.numpy as jnp
from jax.experimental import pallas as pl


def kernel(node_feats, edge_feats, edge_index, W1, b1, ln_gamma, ln_beta, W2, b2):
    raise NotImplementedError("write your pallas kernel here")


# speedup vs baseline: 2.4597x; 2.4597x over previous
import functools

import jax
import jax.numpy as jnp
from jax import lax
from jax.experimental import pallas as pl
from jax.experimental.pallas import tpu as pltpu
from jax.experimental.pallas import tpu_sc as plsc

NODE_DIM = 128
EDGE_DIM = 16
INPUT_DIM = NODE_DIM * 2 + EDGE_DIM


def _sc_gather_body(nf_hbm, row_hbm, col_hbm, gi_hbm, gj_hbm,
                    idx_i, idx_j, rows_i, rows_j, sem_a, sem_b,
                    *, e_per_w, chunk):
    nc = 2
    wid = lax.axis_index("s") * nc + lax.axis_index("c")
    base = wid * e_per_w
    n_iter = e_per_w // chunk

    def body(g, _):
        off = pl.multiple_of(base + g * chunk, 8)
        pltpu.sync_copy(row_hbm.at[pl.ds(off, chunk)], idx_i)
        pltpu.sync_copy(col_hbm.at[pl.ds(off, chunk)], idx_j)
        cp_a = pltpu.async_copy(nf_hbm.at[idx_i], rows_i, sem_a)
        cp_b = pltpu.async_copy(nf_hbm.at[idx_j], rows_j, sem_b)
        cp_a.wait()
        cp_b.wait()
        pltpu.sync_copy(rows_i, gi_hbm.at[pl.ds(off, chunk)])
        pltpu.sync_copy(rows_j, gj_hbm.at[pl.ds(off, chunk)])
        return _

    lax.fori_loop(0, n_iter, body, 0, unroll=False)


def _sc_gather(node_feats, row, col, *, chunk=400):
    n_edges = row.shape[0]
    nw = 32
    e_per_w = n_edges // nw
    mesh = plsc.VectorSubcoreMesh(core_axis_name="c", subcore_axis_name="s")
    out_t = jax.ShapeDtypeStruct((n_edges, NODE_DIM), jnp.float32)
    kern = functools.partial(
        pl.kernel,
        mesh=mesh,
        out_type=[out_t, out_t],
        scratch_types=[
            pltpu.VMEM((chunk,), jnp.int32),
            pltpu.VMEM((chunk,), jnp.int32),
            pltpu.VMEM((chunk, NODE_DIM), jnp.float32),
            pltpu.VMEM((chunk, NODE_DIM), jnp.float32),
            pltpu.SemaphoreType.DMA,
            pltpu.SemaphoreType.DMA,
        ],
    )(functools.partial(_sc_gather_body, e_per_w=e_per_w, chunk=chunk))
    return kern(node_feats, row, col)


def _tc_mlp_body(gi, gj, ef, w1a, w1b, w1c, b1, gam, bet, w2, b2, out):
    h = jnp.dot(gi[...], w1a[...], preferred_element_type=jnp.float32)
    h = h + jnp.dot(gj[...], w1b[...], preferred_element_type=jnp.float32)
    h = h + jnp.dot(ef[...], w1c[...], preferred_element_type=jnp.float32)
    h = h + b1[...]
    mean = jnp.mean(h, axis=-1, keepdims=True)
    hc = h - mean
    var = jnp.mean(hc * hc, axis=-1, keepdims=True)
    hn = hc * lax.rsqrt(var + 1e-5) * gam[...] + bet[...]
    hn = jnp.maximum(hn, 0.0)
    out[...] = (jnp.dot(hn, w2[...], preferred_element_type=jnp.float32)
                + b2[...] + ef[...])


def _tc_mlp(gi, gj, ef, w1a, w1b, w1c, b1, gam, bet, w2, b2, *, tile=2000):
    n_edges = gi.shape[0]
    grid = (n_edges // tile,)

    def edge_spec(width):
        return pl.BlockSpec((tile, width), lambda i: (i, 0))

    def full_spec(a, b):
        return pl.BlockSpec((a, b), lambda i: (0, 0))

    return pl.pallas_call(
        _tc_mlp_body,
        grid=grid,
        in_specs=[
            edge_spec(NODE_DIM),
            edge_spec(NODE_DIM),
            edge_spec(EDGE_DIM),
            full_spec(NODE_DIM, INPUT_DIM),
            full_spec(NODE_DIM, INPUT_DIM),
            full_spec(EDGE_DIM, INPUT_DIM),
            full_spec(1, INPUT_DIM),
            full_spec(1, INPUT_DIM),
            full_spec(1, INPUT_DIM),
            full_spec(INPUT_DIM, EDGE_DIM),
            full_spec(1, EDGE_DIM),
        ],
        out_specs=edge_spec(EDGE_DIM),
        out_shape=jax.ShapeDtypeStruct((n_edges, EDGE_DIM), jnp.float32),
    )(gi, gj, ef, w1a, w1b, w1c, b1, gam, bet, w2, b2)


def kernel(node_feats, edge_feats, edge_index, W1, b1, ln_gamma, ln_beta,
           W2, b2):
    row = edge_index[0].astype(jnp.int32)
    col = edge_index[1].astype(jnp.int32)
    gi, gj = _sc_gather(node_feats, row, col)
    w1t = W1.T
    w1a = w1t[:NODE_DIM]
    w1b = w1t[NODE_DIM:2 * NODE_DIM]
    w1c = w1t[2 * NODE_DIM:]
    return _tc_mlp(
        gi, gj, edge_feats, w1a, w1b, w1c,
        b1.reshape(1, INPUT_DIM),
        ln_gamma.reshape(1, INPUT_DIM),
        ln_beta.reshape(1, INPUT_DIM),
        W2.T,
        b2.reshape(1, EDGE_DIM),
    )
